# Initial kernel scaffold; baseline (speedup 1.0000x reference)
import jax
import jax.numpy as jnp
from jax.experimental import pallas as pl


def kernel(x, edge_index, lin0_W, lin0_b, W1_l1, W1_l2, lin1_W, lin1_b):
    raise NotImplementedError("write your pallas kernel here")


# speedup vs baseline: 4.6247x; 4.6247x over previous
import functools
import math

import jax
import jax.numpy as jnp
from jax import lax
from jax.experimental import pallas as pl
from jax.experimental.pallas import tpu as pltpu
from jax.experimental.pallas import tpu_sc as plsc

_N = 10000
_E = 320000
_D = 128
_ALPHA = 0.1
_THETA = 0.5

_NC = 2
_NS = 16
_NW = _NC * _NS
_EW = _E // _NW
_K = 80
_CH = _EW // _K
_RPT = 624
_RTAIL = _N - _NS * _RPT

_mesh = plsc.VectorSubcoreMesh(core_axis_name="c", subcore_axis_name="s")


@functools.partial(
    pl.kernel,
    out_type=jax.ShapeDtypeStruct((_NC, _N, _D), jnp.float32),
    mesh=_mesh,
    scratch_types=[
        pltpu.VMEM((_K,), jnp.int32),
        pltpu.VMEM((_K,), jnp.int32),
        pltpu.VMEM((_K, _D), jnp.float32),
        pltpu.VMEM_SHARED((_N, _D), jnp.float32),
        pltpu.SemaphoreType.DMA,
    ],
)
def _sc_scatter_add(h_hbm, src_hbm, dst_hbm, zeros_hbm, out_hbm,
                    src_v, dst_v, rows_v, acc_sh, sem):
    c = lax.axis_index("c")
    s = lax.axis_index("s")
    wid = s * _NC + c
    r0 = s * _RPT
    pltpu.sync_copy(zeros_hbm.at[pl.ds(r0, _RPT)], acc_sh.at[pl.ds(r0, _RPT)])

    @pl.when(s == 0)
    def _():
        tb = _NS * _RPT
        pltpu.sync_copy(zeros_hbm.at[pl.ds(tb, _RTAIL)],
                        acc_sh.at[pl.ds(tb, _RTAIL)])

    plsc.subcore_barrier()

    ebase = wid * _EW

    def body(i, carry):
        base = ebase + i * _K
        pltpu.sync_copy(src_hbm.at[pl.ds(base, _K)], src_v)
        pltpu.sync_copy(dst_hbm.at[pl.ds(base, _K)], dst_v)
        pltpu.async_copy(h_hbm.at[src_v], rows_v, sem).wait()
        pltpu.sync_copy(rows_v, acc_sh.at[dst_v], add=True)
        return carry

    lax.fori_loop(0, _CH, body, 0)
    plsc.subcore_barrier()
    pltpu.sync_copy(acc_sh.at[pl.ds(r0, _RPT)], out_hbm.at[c, pl.ds(r0, _RPT)])

    @pl.when(s == 0)
    def _():
        tb = _NS * _RPT
        pltpu.sync_copy(acc_sh.at[pl.ds(tb, _RTAIL)],
                        out_hbm.at[c, pl.ds(tb, _RTAIL)])


_BN = 1000
_G = _N // _BN


def _x0_body(x_ref, w_ref, b_ref, o_ref):
    o_ref[...] = jnp.maximum(
        jnp.dot(x_ref[...], w_ref[...], preferred_element_type=jnp.float32)
        + b_ref[...], 0.0)


_x0_call = pl.pallas_call(
    _x0_body,
    grid=(_G,),
    in_specs=[
        pl.BlockSpec((_BN, _D), lambda i: (i, 0)),
        pl.BlockSpec((_D, _D), lambda i: (0, 0)),
        pl.BlockSpec((1, _D), lambda i: (0, 0)),
    ],
    out_specs=pl.BlockSpec((_BN, _D), lambda i: (i, 0)),
    out_shape=jax.ShapeDtypeStruct((_N, _D), jnp.float32),
)


def _layer_body(beta, p_ref, x0_ref, w_ref, o_ref):
    t = (1.0 - _ALPHA) * (p_ref[0] + p_ref[1]) + _ALPHA * x0_ref[...]
    o_ref[...] = jnp.maximum(
        (1.0 - beta) * t
        + beta * jnp.dot(t, w_ref[...], preferred_element_type=jnp.float32),
        0.0)


_layer1_call = pl.pallas_call(
    functools.partial(_layer_body, math.log(_THETA / 1 + 1.0)),
    grid=(_G,),
    in_specs=[
        pl.BlockSpec((_NC, _BN, _D), lambda i: (0, i, 0)),
        pl.BlockSpec((_BN, _D), lambda i: (i, 0)),
        pl.BlockSpec((_D, _D), lambda i: (0, 0)),
    ],
    out_specs=pl.BlockSpec((_BN, _D), lambda i: (i, 0)),
    out_shape=jax.ShapeDtypeStruct((_N, _D), jnp.float32),
)


def _final_body(beta, p_ref, x0_ref, w1_ref, w2_ref, b2_ref, o_ref):
    t = (1.0 - _ALPHA) * (p_ref[0] + p_ref[1]) + _ALPHA * x0_ref[...]
    h = jnp.maximum(
        (1.0 - beta) * t
        + beta * jnp.dot(t, w1_ref[...], preferred_element_type=jnp.float32),
        0.0)
    z = jnp.dot(h, w2_ref[...], preferred_element_type=jnp.float32) + b2_ref[...]
    z = z - jnp.max(z, axis=-1, keepdims=True)
    o_ref[...] = z - jnp.log(jnp.sum(jnp.exp(z), axis=-1, keepdims=True))


_final_call = pl.pallas_call(
    functools.partial(_final_body, math.log(_THETA / 2 + 1.0)),
    grid=(_G,),
    in_specs=[
        pl.BlockSpec((_NC, _BN, _D), lambda i: (0, i, 0)),
        pl.BlockSpec((_BN, _D), lambda i: (i, 0)),
        pl.BlockSpec((_D, _D), lambda i: (0, 0)),
        pl.BlockSpec((_D, _D), lambda i: (0, 0)),
        pl.BlockSpec((1, _D), lambda i: (0, 0)),
    ],
    out_specs=pl.BlockSpec((_BN, _D), lambda i: (i, 0)),
    out_shape=jax.ShapeDtypeStruct((_N, _D), jnp.float32),
)


def kernel(x, edge_index, lin0_W, lin0_b, W1_l1, W1_l2, lin1_W, lin1_b):
    src = edge_index[0]
    dst = edge_index[1]
    zeros = jnp.zeros((_N, _D), jnp.float32)
    x0 = _x0_call(x, lin0_W.T, lin0_b.reshape(1, _D))
    p1 = _sc_scatter_add(x0, src, dst, zeros)
    h1 = _layer1_call(p1, x0, W1_l1)
    p2 = _sc_scatter_add(h1, src, dst, zeros)
    return _final_call(p2, x0, W1_l2, lin1_W.T, lin1_b.reshape(1, _D))
